# baseline (speedup 1.0000x reference)
import functools

import jax
import jax.numpy as jnp
from jax.experimental import pallas as pl
from jax.experimental.pallas import tpu as pltpu

_EPS = 1e-5
_VP = 32


def _rshift(a, s):
    n, c = a.shape
    return jnp.concatenate([a[s:], jnp.zeros((s, c), a.dtype)], axis=0)


def _p1_kernel(x_ref, au_ref, wv_ref, h_ref, st_ref, *, T, V, C):
    xb = x_ref[0]
    xcv = jnp.transpose(xb, (0, 2, 1))
    xm = xcv.reshape(T * C, V)
    q = jnp.dot(xm, au_ref[...],
                preferred_element_type=jnp.float32)
    q3 = jnp.transpose(q.reshape(T, C, 3 * _VP), (0, 2, 1))
    qcat = jnp.concatenate(
        [q3[:, 0:_VP, :].reshape(T * _VP, C),
         q3[:, _VP:2 * _VP, :].reshape(T * _VP, C),
         q3[:, 2 * _VP:3 * _VP, :].reshape(T * _VP, C)], axis=1)
    hm = jnp.dot(qcat, wv_ref[...],
                 preferred_element_type=jnp.float32)
    h_ref[0] = hm
    s = jnp.sum(hm, axis=0, keepdims=True)
    sq = jnp.sum(hm * hm, axis=0, keepdims=True)
    part = jnp.concatenate([s, sq, jnp.zeros((6, C), jnp.float32)], axis=0)

    @pl.when(pl.program_id(0) == 0)
    def _():
        st_ref[...] = jnp.zeros_like(st_ref)

    st_ref[...] += part


def _p2_kernel(h_ref, ss_ref, wt_ref, h2_ref, st_ref, *, T, V, C):
    hm = h_ref[0]
    sc = ss_ref[0:1, :]
    sh = ss_ref[1:2, :]
    g = jnp.maximum(hm * sc + sh, 0.0)
    rid = jax.lax.broadcasted_iota(jnp.int32, (T * _VP, 1), 0)
    g = jnp.where(jnp.bitwise_and(rid, _VP - 1) < V, g, 0.0)
    z = jnp.dot(g, wt_ref[...], preferred_element_type=jnp.float32)
    zp = jnp.concatenate([jnp.zeros((4 * _VP, C), jnp.float32), z], axis=0)
    f = zp + _rshift(zp, _VP)
    f = f + _rshift(f, 2 * _VP)
    f = f + _rshift(f, 4 * _VP)
    f = f + _rshift(zp, 8 * _VP)
    h2 = f[: T * _VP]
    h2_ref[0] = h2
    s = jnp.sum(h2, axis=0, keepdims=True)
    sq = jnp.sum(h2 * h2, axis=0, keepdims=True)
    part = jnp.concatenate([s, sq, jnp.zeros((6, C), jnp.float32)], axis=0)

    @pl.when(pl.program_id(0) == 0)
    def _():
        st_ref[...] = jnp.zeros_like(st_ref)

    st_ref[...] += part


def _p3_kernel(h2_ref, ss_ref, out_ref, *, T, V, C):
    hm = h2_ref[0]
    sc = ss_ref[0:1, :]
    sh = ss_ref[1:2, :]
    o = jnp.maximum(hm * sc + sh, 0.0)
    out_ref[0] = o.reshape(T, _VP, C)[:, :V, :]


def _bn_affine(st, n, gamma, beta):
    mean = st[0] / n
    var = st[1] / n - mean * mean
    inv = gamma * jax.lax.rsqrt(var + _EPS)
    return jnp.stack([inv, beta - mean * inv])


def kernel(x, adj, edge_importance, W1, b1, W2, b2, W3, b3, Wt, bt, gamma, beta):
    B, T, V, C = x.shape
    f32 = jnp.float32
    params = pltpu.CompilerParams(dimension_semantics=("arbitrary",))
    small = lambda shp: pl.BlockSpec(shp, lambda b: (0,) * len(shp))

    A = adj * edge_importance
    At = jnp.transpose(A, (0, 2, 1))
    Au = jnp.pad(At, ((0, 0), (0, 0), (0, _VP - V)))
    Au = jnp.transpose(Au, (1, 0, 2)).reshape(V, 3 * _VP)
    Wv = jnp.concatenate([W1, W2, W3], axis=0)

    p1 = pl.pallas_call(
        functools.partial(_p1_kernel, T=T, V=V, C=C),
        grid=(B,),
        in_specs=[
            pl.BlockSpec((1, T, V, C), lambda b: (b, 0, 0, 0)),
            small((V, 3 * _VP)), small((3 * C, C)),
        ],
        out_specs=[
            pl.BlockSpec((1, T * _VP, C), lambda b: (b, 0, 0)),
            small((8, C)),
        ],
        out_shape=[
            jax.ShapeDtypeStruct((B, T * _VP, C), f32),
            jax.ShapeDtypeStruct((8, C), f32),
        ],
        compiler_params=params,
    )
    h, st1 = p1(x, Au, Wv)
    ss1 = _bn_affine(st1, float(B * T * V), gamma, beta)

    p2 = pl.pallas_call(
        functools.partial(_p2_kernel, T=T, V=V, C=C),
        grid=(B,),
        in_specs=[
            pl.BlockSpec((1, T * _VP, C), lambda b: (b, 0, 0)),
            small((2, C)), small((C, C)),
        ],
        out_specs=[
            pl.BlockSpec((1, T * _VP, C), lambda b: (b, 0, 0)),
            small((8, C)),
        ],
        out_shape=[
            jax.ShapeDtypeStruct((B, T * _VP, C), f32),
            jax.ShapeDtypeStruct((8, C), f32),
        ],
        compiler_params=params,
    )
    h2, st2 = p2(h, ss1, Wt)
    ss2 = _bn_affine(st2, float(B * T * V), gamma, beta)

    p3 = pl.pallas_call(
        functools.partial(_p3_kernel, T=T, V=V, C=C),
        grid=(B,),
        in_specs=[
            pl.BlockSpec((1, T * _VP, C), lambda b: (b, 0, 0)),
            small((2, C)),
        ],
        out_specs=pl.BlockSpec((1, T, V, C), lambda b: (b, 0, 0, 0)),
        out_shape=jax.ShapeDtypeStruct((B, T, V, C), f32),
        compiler_params=params,
    )
    return p3(h2, ss2)
